# baseline (speedup 1.0000x reference)
import functools

import jax
import jax.numpy as jnp
from jax import lax
from jax.experimental import pallas as pl
from jax.experimental.pallas import tpu as pltpu
from jax.experimental.pallas import tpu_sc as plsc

_LANES = 16


def _router_body(rw_ref, rb_ref, x_ref, d_ref):
    rdiff = rw_ref[0:1, :] - rw_ref[1:2, :]
    bdiff = rb_ref[0] - rb_ref[1]
    d = jax.lax.dot_general(
        rdiff, x_ref[...], (((1,), (1,)), ((), ())),
        preferred_element_type=jnp.float32)
    d_ref[0] = d + bdiff


def _router_d(xf, router_w, router_b, n_blk):
    bs, h = xf.shape
    blk = bs // n_blk
    d3 = pl.pallas_call(
        _router_body,
        grid=(n_blk,),
        in_specs=[
            pl.BlockSpec((router_w.shape[0], h), lambda i: (0, 0)),
            pl.BlockSpec(memory_space=pltpu.SMEM),
            pl.BlockSpec((blk, h), lambda i: (i, 0)),
        ],
        out_specs=pl.BlockSpec((1, 1, blk), lambda i: (i, 0, 0)),
        out_shape=jax.ShapeDtypeStruct((n_blk, 1, blk), jnp.float32),
    )(router_w, router_b, xf)
    return d3


def _make_sc_select(bs, h):
    mesh = plsc.VectorSubcoreMesh(core_axis_name="c", subcore_axis_name="s")
    n_chunks = bs // _LANES

    @functools.partial(
        pl.kernel,
        mesh=mesh,
        out_type=(
            jax.ShapeDtypeStruct((_LANES,), jnp.int32),
            jax.ShapeDtypeStruct((_LANES,), jnp.float32),
            jax.ShapeDtypeStruct((_LANES, h), jnp.float32),
        ),
        scratch_types=[
            pltpu.VMEM((bs,), jnp.float32),
            pltpu.VMEM((_LANES,), jnp.int32),
            pltpu.VMEM((_LANES,), jnp.float32),
            pltpu.VMEM((_LANES, h), jnp.float32),
            pltpu.VMEM((2 * _LANES,), jnp.float32),
            pltpu.VMEM((2 * _LANES,), jnp.int32),
            pltpu.SemaphoreType.DMA,
        ],
    )
    def sc_select(d_hbm, x_hbm, tok_out, gate_out, rows_out,
                  d_v, tok_v, gate_v, rows_v, buff, bufi, sem):
        cid = lax.axis_index("c")
        sid = lax.axis_index("s")

        @pl.when(jnp.logical_and(cid == 0, sid == 0))
        def _():
            pltpu.sync_copy(d_hbm, d_v)
            lane = lax.iota(jnp.int32, _LANES)
            neg = jnp.float32(-3.0e38)
            pos = jnp.float32(3.0e38)

            def rot_f(v, k):
                buff[pl.ds(0, _LANES)] = v
                buff[pl.ds(_LANES, _LANES)] = v
                return buff[pl.ds(k, _LANES)]

            def rot_i(v, k):
                bufi[pl.ds(0, _LANES)] = v
                bufi[pl.ds(_LANES, _LANES)] = v
                return bufi[pl.ds(k, _LANES)]

            def xreduce(v, i, is_max):
                for sh in (1, 2, 4, 8):
                    v2 = rot_f(v, sh)
                    i2 = rot_i(i, sh)
                    if is_max:
                        t = (v2 > v) | ((v2 == v) & (i2 < i))
                    else:
                        t = (v2 < v) | ((v2 == v) & (i2 < i))
                    v = jnp.where(t, v2, v)
                    i = jnp.where(t, i2, i)
                return v, i

            def body(i, carry):
                mx1, im1, mx2, im2, mn1, jn1, mn2, jn2 = carry
                v = d_v[pl.ds(i * _LANES, _LANES)]
                idx = lane + i * _LANES
                gt1 = v > mx1
                gt2 = v > mx2
                mx2 = jnp.where(gt1, mx1, jnp.where(gt2, v, mx2))
                im2 = jnp.where(gt1, im1, jnp.where(gt2, idx, im2))
                mx1 = jnp.where(gt1, v, mx1)
                im1 = jnp.where(gt1, idx, im1)
                lt1 = v < mn1
                lt2 = v < mn2
                mn2 = jnp.where(lt1, mn1, jnp.where(lt2, v, mn2))
                jn2 = jnp.where(lt1, jn1, jnp.where(lt2, idx, jn2))
                mn1 = jnp.where(lt1, v, mn1)
                jn1 = jnp.where(lt1, idx, jn1)
                return mx1, im1, mx2, im2, mn1, jn1, mn2, jn2

            zi = jnp.zeros((_LANES,), jnp.int32)
            mx1, im1, mx2, im2, mn1, jn1, mn2, jn2 = lax.fori_loop(
                0, n_chunks, body,
                (jnp.full((_LANES,), neg), zi, jnp.full((_LANES,), neg), zi,
                 jnp.full((_LANES,), pos), zi, jnp.full((_LANES,), pos), zi))
            m1, i1 = xreduce(mx1, im1, True)
            n1, j1 = xreduce(mn1, jn1, False)
            winx = (mx1 == m1) & (im1 == i1)
            m2, i2 = xreduce(jnp.where(winx, mx2, mx1),
                             jnp.where(winx, im2, im1), True)
            winn = (mn1 == n1) & (jn1 == j1)
            n2, j2 = xreduce(jnp.where(winn, mn2, mn1),
                             jnp.where(winn, jn2, jn1), False)

            z = jnp.where(lane == 0, m1,
                          jnp.where(lane == 1, -n1,
                                    jnp.where(lane == 2, m2,
                                              jnp.where(lane == 3, -n2, 0.0))))
            gates = 1.0 / (1.0 + jnp.exp(-z))
            gates = jnp.where(lane < 4, gates, 0.0)
            toks = jnp.where(lane == 0, i1,
                             jnp.where(lane == 1, i2,
                                       jnp.where(lane == 2, j1,
                                                 jnp.where(lane == 3, j2, 0))))

            g2 = gates
            for k in (1, 2, 3, 13, 14, 15):
                tk = rot_i(toks, k)
                gk = rot_f(gates, k)
                g2 = g2 + jnp.where(toks == tk, gk, 0.0)
            gfin = jnp.where(lane < 4, g2, 0.0)
            tstore = toks

            tok_v[...] = tstore
            gate_v[...] = gfin
            pltpu.sync_copy(tok_v, tok_out)
            pltpu.sync_copy(gate_v, gate_out)
            pltpu.async_copy(x_hbm.at[tok_v], rows_v, sem).wait()
            pltpu.sync_copy(rows_v, rows_out)

    return sc_select


def _zero_body(d_ref, y_ref):
    y_ref[...] = jnp.zeros(y_ref.shape, y_ref.dtype)


def _zero_fill(d3, bs, h, n_blk):
    blk = bs // n_blk
    return pl.pallas_call(
        _zero_body,
        grid=(n_blk,),
        in_specs=[pl.BlockSpec(memory_space=pl.ANY)],
        out_specs=pl.BlockSpec((blk, h), lambda j: (j, 0)),
        out_shape=jax.ShapeDtypeStruct((bs, h), jnp.float32),
    )(d3)


def _mm_body(gate_ref, rows_ref, w_ref, b_ref, out_ref):
    e = jax.lax.dot_general(
        rows_ref[...], w_ref[...], (((1,), (1,)), ((), ())),
        preferred_element_type=jnp.float32) + b_ref[...]
    ridx = jax.lax.broadcasted_iota(jnp.int32, (_LANES, 1), 0)
    scale = jnp.where(ridx == 0, gate_ref[0],
                      jnp.where(ridx == 1, gate_ref[1],
                                jnp.where(ridx == 2, gate_ref[2],
                                          jnp.where(ridx == 3, gate_ref[3], 0.0))))
    out_ref[...] = e * scale


def _expert_mm(gates, rows, expert_w, expert_b, n_chunk=8):
    h = expert_w.shape[0]
    csz = h // n_chunk
    return pl.pallas_call(
        _mm_body,
        grid=(n_chunk,),
        in_specs=[
            pl.BlockSpec(memory_space=pltpu.SMEM),
            pl.BlockSpec((_LANES, h), lambda j: (0, 0)),
            pl.BlockSpec((csz, h), lambda j: (j, 0)),
            pl.BlockSpec((1, csz), lambda j: (0, j)),
        ],
        out_specs=pl.BlockSpec((_LANES, csz), lambda j: (0, j)),
        out_shape=jax.ShapeDtypeStruct((_LANES, h), jnp.float32),
    )(gates, rows, expert_w, expert_b.reshape(1, h))


def _scatter_body(tok_ref, eout_ref, yin_ref, yout_ref, sem):
    cps = [pltpu.make_async_copy(
        eout_ref.at[pl.ds(j, 1), :],
        yout_ref.at[pl.ds(tok_ref[j], 1), :],
        sem) for j in range(4)]
    for cp in cps:
        cp.start()
    for cp in cps:
        cp.wait()


def _scatter_rows(toks, eout, yzero):
    bs, h = yzero.shape
    return pl.pallas_call(
        _scatter_body,
        grid_spec=pltpu.PrefetchScalarGridSpec(
            num_scalar_prefetch=1,
            grid=(1,),
            in_specs=[
                pl.BlockSpec((_LANES, h), lambda j, tok_ref: (0, 0)),
                pl.BlockSpec(memory_space=pl.ANY),
            ],
            out_specs=pl.BlockSpec(memory_space=pl.ANY),
            scratch_shapes=[pltpu.SemaphoreType.DMA],
        ),
        out_shape=jax.ShapeDtypeStruct((bs, h), jnp.float32),
        input_output_aliases={2: 0},
    )(toks, eout, yzero)


def kernel(x, router_w, router_b, expert_w, expert_b):
    b, s, h = x.shape
    xf = x.reshape(-1, h)
    bs = xf.shape[0]
    n_blk = 2
    d3 = _router_d(xf, router_w, router_b, n_blk)
    toks, gates, rows = _make_sc_select(bs, h)(d3.reshape(bs), xf)
    yzero = _zero_fill(d3, bs, h, 4)
    eout = _expert_mm(gates, rows, expert_w, expert_b)
    y = _scatter_rows(toks, eout, yzero)
    return y.reshape(b, s, h)


# speedup vs baseline: 1.0151x; 1.0151x over previous
import functools

import jax
import jax.numpy as jnp
from jax import lax
from jax.experimental import pallas as pl
from jax.experimental.pallas import tpu as pltpu
from jax.experimental.pallas import tpu_sc as plsc

_LANES = 16


def _router_body(rw_ref, rb_ref, x_ref, d_ref):
    rdiff = rw_ref[0:1, :] - rw_ref[1:2, :]
    bdiff = rb_ref[0] - rb_ref[1]
    d = jax.lax.dot_general(
        rdiff, x_ref[...], (((1,), (1,)), ((), ())),
        preferred_element_type=jnp.float32)
    d_ref[0] = d + bdiff


def _router_d(xf, router_w, router_b, n_blk):
    bs, h = xf.shape
    blk = bs // n_blk
    d3 = pl.pallas_call(
        _router_body,
        grid=(n_blk,),
        in_specs=[
            pl.BlockSpec((router_w.shape[0], h), lambda i: (0, 0)),
            pl.BlockSpec(memory_space=pltpu.SMEM),
            pl.BlockSpec((blk, h), lambda i: (i, 0)),
        ],
        out_specs=pl.BlockSpec((1, 1, blk), lambda i: (i, 0, 0)),
        out_shape=jax.ShapeDtypeStruct((n_blk, 1, blk), jnp.float32),
    )(router_w, router_b, xf)
    return d3


def _make_sc_select(bs, h):
    mesh = plsc.VectorSubcoreMesh(core_axis_name="c", subcore_axis_name="s")
    n_chunks = bs // _LANES

    @functools.partial(
        pl.kernel,
        mesh=mesh,
        out_type=(
            jax.ShapeDtypeStruct((_LANES,), jnp.int32),
            jax.ShapeDtypeStruct((_LANES,), jnp.float32),
            jax.ShapeDtypeStruct((_LANES, h), jnp.float32),
        ),
        scratch_types=[
            pltpu.VMEM((bs,), jnp.float32),
            pltpu.VMEM((_LANES,), jnp.int32),
            pltpu.VMEM((_LANES,), jnp.float32),
            pltpu.VMEM((_LANES, h), jnp.float32),
            pltpu.VMEM((2 * _LANES,), jnp.float32),
            pltpu.VMEM((2 * _LANES,), jnp.int32),
            pltpu.SemaphoreType.DMA,
        ],
    )
    def sc_select(d_hbm, x_hbm, tok_out, gate_out, rows_out,
                  d_v, tok_v, gate_v, rows_v, buff, bufi, sem):
        cid = lax.axis_index("c")
        sid = lax.axis_index("s")

        @pl.when(jnp.logical_and(cid == 0, sid == 0))
        def _():
            pltpu.sync_copy(d_hbm, d_v)
            lane = lax.iota(jnp.int32, _LANES)
            neg = jnp.float32(-3.0e38)
            pos = jnp.float32(3.0e38)

            def rot_f(v, k):
                buff[pl.ds(0, _LANES)] = v
                buff[pl.ds(_LANES, _LANES)] = v
                return buff[pl.ds(k, _LANES)]

            def rot_i(v, k):
                bufi[pl.ds(0, _LANES)] = v
                bufi[pl.ds(_LANES, _LANES)] = v
                return bufi[pl.ds(k, _LANES)]

            def xreduce(v, i, is_max):
                for sh in (1, 2, 4, 8):
                    v2 = rot_f(v, sh)
                    i2 = rot_i(i, sh)
                    if is_max:
                        t = (v2 > v) | ((v2 == v) & (i2 < i))
                    else:
                        t = (v2 < v) | ((v2 == v) & (i2 < i))
                    v = jnp.where(t, v2, v)
                    i = jnp.where(t, i2, i)
                return v, i

            def body(i, carry):
                mx1, im1, mx2, im2, mn1, jn1, mn2, jn2 = carry
                v = d_v[pl.ds(i * _LANES, _LANES)]
                idx = lane + i * _LANES
                gt1 = v > mx1
                gt2 = v > mx2
                mx2 = jnp.where(gt1, mx1, jnp.where(gt2, v, mx2))
                im2 = jnp.where(gt1, im1, jnp.where(gt2, idx, im2))
                mx1 = jnp.where(gt1, v, mx1)
                im1 = jnp.where(gt1, idx, im1)
                lt1 = v < mn1
                lt2 = v < mn2
                mn2 = jnp.where(lt1, mn1, jnp.where(lt2, v, mn2))
                jn2 = jnp.where(lt1, jn1, jnp.where(lt2, idx, jn2))
                mn1 = jnp.where(lt1, v, mn1)
                jn1 = jnp.where(lt1, idx, jn1)
                return mx1, im1, mx2, im2, mn1, jn1, mn2, jn2

            zi = jnp.zeros((_LANES,), jnp.int32)
            mx1, im1, mx2, im2, mn1, jn1, mn2, jn2 = lax.fori_loop(
                0, n_chunks, body,
                (jnp.full((_LANES,), neg), zi, jnp.full((_LANES,), neg), zi,
                 jnp.full((_LANES,), pos), zi, jnp.full((_LANES,), pos), zi))
            m1, i1 = xreduce(mx1, im1, True)
            n1, j1 = xreduce(mn1, jn1, False)
            winx = (mx1 == m1) & (im1 == i1)
            m2, i2 = xreduce(jnp.where(winx, mx2, mx1),
                             jnp.where(winx, im2, im1), True)
            winn = (mn1 == n1) & (jn1 == j1)
            n2, j2 = xreduce(jnp.where(winn, mn2, mn1),
                             jnp.where(winn, jn2, jn1), False)

            z = jnp.where(lane == 0, m1,
                          jnp.where(lane == 1, -n1,
                                    jnp.where(lane == 2, m2,
                                              jnp.where(lane == 3, -n2, 0.0))))
            gates = 1.0 / (1.0 + jnp.exp(-z))
            gates = jnp.where(lane < 4, gates, 0.0)
            toks = jnp.where(lane == 0, i1,
                             jnp.where(lane == 1, i2,
                                       jnp.where(lane == 2, j1,
                                                 jnp.where(lane == 3, j2, 0))))

            g2 = gates
            for k in (1, 2, 3, 13, 14, 15):
                tk = rot_i(toks, k)
                gk = rot_f(gates, k)
                g2 = g2 + jnp.where(toks == tk, gk, 0.0)
            gfin = jnp.where(lane < 4, g2, 0.0)
            tstore = toks

            tok_v[...] = tstore
            gate_v[...] = gfin
            pltpu.sync_copy(tok_v, tok_out)
            pltpu.sync_copy(gate_v, gate_out)
            pltpu.async_copy(x_hbm.at[tok_v], rows_v, sem).wait()
            pltpu.sync_copy(rows_v, rows_out)

    return sc_select


def _zero_body(d_ref, y_ref):
    y_ref[...] = jnp.zeros(y_ref.shape, y_ref.dtype)


def _zero_fill(d3, bs, h, n_blk):
    blk = bs // n_blk
    return pl.pallas_call(
        _zero_body,
        grid=(n_blk,),
        in_specs=[pl.BlockSpec(memory_space=pl.ANY)],
        out_specs=pl.BlockSpec((blk, h), lambda j: (j, 0)),
        out_shape=jax.ShapeDtypeStruct((bs, h), jnp.float32),
    )(d3)


def _mm_body(gate_ref, rows_ref, w_ref, b_ref, out_ref):
    e = jax.lax.dot_general(
        rows_ref[...], w_ref[...], (((1,), (1,)), ((), ())),
        preferred_element_type=jnp.float32) + b_ref[...]
    ridx = jax.lax.broadcasted_iota(jnp.int32, (_LANES, 1), 0)
    scale = jnp.where(ridx == 0, gate_ref[0],
                      jnp.where(ridx == 1, gate_ref[1],
                                jnp.where(ridx == 2, gate_ref[2],
                                          jnp.where(ridx == 3, gate_ref[3], 0.0))))
    out_ref[...] = e * scale


def _expert_mm(gates, rows, expert_w, expert_b, n_chunk=8):
    h = expert_w.shape[0]
    csz = h // n_chunk
    return pl.pallas_call(
        _mm_body,
        grid=(n_chunk,),
        in_specs=[
            pl.BlockSpec(memory_space=pltpu.SMEM),
            pl.BlockSpec((_LANES, h), lambda j: (0, 0)),
            pl.BlockSpec((csz, h), lambda j: (j, 0)),
            pl.BlockSpec((1, csz), lambda j: (0, j)),
        ],
        out_specs=pl.BlockSpec((_LANES, csz), lambda j: (0, j)),
        out_shape=jax.ShapeDtypeStruct((_LANES, h), jnp.float32),
    )(gates, rows, expert_w, expert_b.reshape(1, h))


def _scatter_body(tok_ref, eout_ref, yin_ref, yout_ref, sem):
    cps = [pltpu.make_async_copy(
        eout_ref.at[pl.ds(j, 1), :],
        yout_ref.at[pl.ds(tok_ref[j], 1), :],
        sem) for j in range(4)]
    for cp in cps:
        cp.start()
    for cp in cps:
        cp.wait()


def _scatter_rows(toks, eout, yzero):
    bs, h = yzero.shape
    return pl.pallas_call(
        _scatter_body,
        grid_spec=pltpu.PrefetchScalarGridSpec(
            num_scalar_prefetch=1,
            grid=(1,),
            in_specs=[
                pl.BlockSpec((_LANES, h), lambda j, tok_ref: (0, 0)),
                pl.BlockSpec(memory_space=pl.ANY),
            ],
            out_specs=pl.BlockSpec(memory_space=pl.ANY),
            scratch_shapes=[pltpu.SemaphoreType.DMA],
        ),
        out_shape=jax.ShapeDtypeStruct((bs, h), jnp.float32),
        input_output_aliases={2: 0},
    )(toks, eout, yzero)


def kernel(x, router_w, router_b, expert_w, expert_b):
    b, s, h = x.shape
    xf = x.reshape(-1, h)
    bs = xf.shape[0]
    n_blk = 4
    d3 = _router_d(xf, router_w, router_b, n_blk)
    toks, gates, rows = _make_sc_select(bs, h)(d3.reshape(bs), xf)
    yzero = _zero_fill(d3, bs, h, 4)
    eout = _expert_mm(gates, rows, expert_w, expert_b)
    y = _scatter_rows(toks, eout, yzero)
    return y.reshape(b, s, h)


# speedup vs baseline: 1.0450x; 1.0295x over previous
import functools

import jax
import jax.numpy as jnp
from jax import lax
from jax.experimental import pallas as pl
from jax.experimental.pallas import tpu as pltpu
from jax.experimental.pallas import tpu_sc as plsc

_LANES = 16


def _router_body(rw_ref, rb_ref, x_ref, d_ref):
    xb = x_ref[...].astype(jnp.bfloat16)
    rwb = rw_ref[...].astype(jnp.bfloat16)
    l = jax.lax.dot_general(
        rwb, xb, (((1,), (1,)), ((), ())),
        preferred_element_type=jnp.float32)
    d_ref[0] = (l[0:1, :] - l[1:2, :]) + (rb_ref[0] - rb_ref[1])


def _router_d(xf, router_w, router_b, n_blk):
    bs, h = xf.shape
    blk = bs // n_blk
    d3 = pl.pallas_call(
        _router_body,
        grid=(n_blk,),
        in_specs=[
            pl.BlockSpec((router_w.shape[0], h), lambda i: (0, 0)),
            pl.BlockSpec(memory_space=pltpu.SMEM),
            pl.BlockSpec((blk, h), lambda i: (i, 0)),
        ],
        out_specs=pl.BlockSpec((1, 1, blk), lambda i: (i, 0, 0)),
        out_shape=jax.ShapeDtypeStruct((n_blk, 1, blk), jnp.float32),
    )(router_w, router_b, xf)
    return d3


def _make_sc_select(bs, h):
    mesh = plsc.VectorSubcoreMesh(core_axis_name="c", subcore_axis_name="s")
    n_chunks = bs // _LANES

    @functools.partial(
        pl.kernel,
        mesh=mesh,
        out_type=(
            jax.ShapeDtypeStruct((_LANES,), jnp.int32),
            jax.ShapeDtypeStruct((_LANES,), jnp.float32),
            jax.ShapeDtypeStruct((_LANES, h), jnp.float32),
        ),
        scratch_types=[
            pltpu.VMEM((bs,), jnp.float32),
            pltpu.VMEM((_LANES,), jnp.int32),
            pltpu.VMEM((_LANES,), jnp.float32),
            pltpu.VMEM((_LANES, h), jnp.float32),
            pltpu.VMEM((2 * _LANES,), jnp.float32),
            pltpu.VMEM((2 * _LANES,), jnp.int32),
            pltpu.SemaphoreType.DMA,
        ],
    )
    def sc_select(d_hbm, x_hbm, tok_out, gate_out, rows_out,
                  d_v, tok_v, gate_v, rows_v, buff, bufi, sem):
        cid = lax.axis_index("c")
        sid = lax.axis_index("s")

        @pl.when(jnp.logical_and(cid == 0, sid == 0))
        def _():
            pltpu.sync_copy(d_hbm, d_v)
            lane = lax.iota(jnp.int32, _LANES)
            neg = jnp.float32(-3.0e38)
            pos = jnp.float32(3.0e38)

            def rot_f(v, k):
                buff[pl.ds(0, _LANES)] = v
                buff[pl.ds(_LANES, _LANES)] = v
                return buff[pl.ds(k, _LANES)]

            def rot_i(v, k):
                bufi[pl.ds(0, _LANES)] = v
                bufi[pl.ds(_LANES, _LANES)] = v
                return bufi[pl.ds(k, _LANES)]

            def xreduce(v, i, is_max):
                for sh in (1, 2, 4, 8):
                    v2 = rot_f(v, sh)
                    i2 = rot_i(i, sh)
                    if is_max:
                        t = (v2 > v) | ((v2 == v) & (i2 < i))
                    else:
                        t = (v2 < v) | ((v2 == v) & (i2 < i))
                    v = jnp.where(t, v2, v)
                    i = jnp.where(t, i2, i)
                return v, i

            def body(i, carry):
                mx1, im1, mx2, im2, mn1, jn1, mn2, jn2 = carry
                v = d_v[pl.ds(i * _LANES, _LANES)]
                idx = lane + i * _LANES
                gt1 = v > mx1
                gt2 = v > mx2
                mx2 = jnp.where(gt1, mx1, jnp.where(gt2, v, mx2))
                im2 = jnp.where(gt1, im1, jnp.where(gt2, idx, im2))
                mx1 = jnp.where(gt1, v, mx1)
                im1 = jnp.where(gt1, idx, im1)
                lt1 = v < mn1
                lt2 = v < mn2
                mn2 = jnp.where(lt1, mn1, jnp.where(lt2, v, mn2))
                jn2 = jnp.where(lt1, jn1, jnp.where(lt2, idx, jn2))
                mn1 = jnp.where(lt1, v, mn1)
                jn1 = jnp.where(lt1, idx, jn1)
                return mx1, im1, mx2, im2, mn1, jn1, mn2, jn2

            zi = jnp.zeros((_LANES,), jnp.int32)
            mx1, im1, mx2, im2, mn1, jn1, mn2, jn2 = lax.fori_loop(
                0, n_chunks, body,
                (jnp.full((_LANES,), neg), zi, jnp.full((_LANES,), neg), zi,
                 jnp.full((_LANES,), pos), zi, jnp.full((_LANES,), pos), zi))
            m1, i1 = xreduce(mx1, im1, True)
            n1, j1 = xreduce(mn1, jn1, False)
            winx = (mx1 == m1) & (im1 == i1)
            m2, i2 = xreduce(jnp.where(winx, mx2, mx1),
                             jnp.where(winx, im2, im1), True)
            winn = (mn1 == n1) & (jn1 == j1)
            n2, j2 = xreduce(jnp.where(winn, mn2, mn1),
                             jnp.where(winn, jn2, jn1), False)

            z = jnp.where(lane == 0, m1,
                          jnp.where(lane == 1, -n1,
                                    jnp.where(lane == 2, m2,
                                              jnp.where(lane == 3, -n2, 0.0))))
            gates = 1.0 / (1.0 + jnp.exp(-z))
            gates = jnp.where(lane < 4, gates, 0.0)
            toks = jnp.where(lane == 0, i1,
                             jnp.where(lane == 1, i2,
                                       jnp.where(lane == 2, j1,
                                                 jnp.where(lane == 3, j2, 0))))

            g2 = gates
            for k in (1, 2, 3, 13, 14, 15):
                tk = rot_i(toks, k)
                gk = rot_f(gates, k)
                g2 = g2 + jnp.where(toks == tk, gk, 0.0)
            gfin = jnp.where(lane < 4, g2, 0.0)
            tstore = toks

            tok_v[...] = tstore
            gate_v[...] = gfin
            pltpu.sync_copy(tok_v, tok_out)
            pltpu.sync_copy(gate_v, gate_out)
            pltpu.async_copy(x_hbm.at[tok_v], rows_v, sem).wait()
            pltpu.sync_copy(rows_v, rows_out)

    return sc_select


def _zero_body(d_ref, y_ref):
    y_ref[...] = jnp.zeros(y_ref.shape, y_ref.dtype)


def _zero_fill(d3, bs, h, n_blk):
    blk = bs // n_blk
    return pl.pallas_call(
        _zero_body,
        grid=(n_blk,),
        in_specs=[pl.BlockSpec(memory_space=pl.ANY)],
        out_specs=pl.BlockSpec((blk, h), lambda j: (j, 0)),
        out_shape=jax.ShapeDtypeStruct((bs, h), jnp.float32),
    )(d3)


def _mm_body(gate_ref, rows_ref, w_ref, b_ref, out_ref):
    e = jax.lax.dot_general(
        rows_ref[...].astype(jnp.bfloat16), w_ref[...].astype(jnp.bfloat16),
        (((1,), (1,)), ((), ())),
        preferred_element_type=jnp.float32) + b_ref[...]
    ridx = jax.lax.broadcasted_iota(jnp.int32, (_LANES, 1), 0)
    scale = jnp.where(ridx == 0, gate_ref[0],
                      jnp.where(ridx == 1, gate_ref[1],
                                jnp.where(ridx == 2, gate_ref[2],
                                          jnp.where(ridx == 3, gate_ref[3], 0.0))))
    out_ref[...] = e * scale


def _expert_mm(gates, rows, expert_w, expert_b, n_chunk=4):
    h = expert_w.shape[0]
    csz = h // n_chunk
    return pl.pallas_call(
        _mm_body,
        grid=(n_chunk,),
        in_specs=[
            pl.BlockSpec(memory_space=pltpu.SMEM),
            pl.BlockSpec((_LANES, h), lambda j: (0, 0)),
            pl.BlockSpec((csz, h), lambda j: (j, 0)),
            pl.BlockSpec((1, csz), lambda j: (0, j)),
        ],
        out_specs=pl.BlockSpec((_LANES, csz), lambda j: (0, j)),
        out_shape=jax.ShapeDtypeStruct((_LANES, h), jnp.float32),
    )(gates, rows, expert_w, expert_b.reshape(1, h))


def _scatter_body(tok_ref, eout_ref, yin_ref, yout_ref, sem):
    cps = [pltpu.make_async_copy(
        eout_ref.at[pl.ds(j, 1), :],
        yout_ref.at[pl.ds(tok_ref[j], 1), :],
        sem) for j in range(4)]
    for cp in cps:
        cp.start()
    for cp in cps:
        cp.wait()


def _scatter_rows(toks, eout, yzero):
    bs, h = yzero.shape
    return pl.pallas_call(
        _scatter_body,
        grid_spec=pltpu.PrefetchScalarGridSpec(
            num_scalar_prefetch=1,
            grid=(1,),
            in_specs=[
                pl.BlockSpec((_LANES, h), lambda j, tok_ref: (0, 0)),
                pl.BlockSpec(memory_space=pl.ANY),
            ],
            out_specs=pl.BlockSpec(memory_space=pl.ANY),
            scratch_shapes=[pltpu.SemaphoreType.DMA],
        ),
        out_shape=jax.ShapeDtypeStruct((bs, h), jnp.float32),
        input_output_aliases={2: 0},
    )(toks, eout, yzero)


def kernel(x, router_w, router_b, expert_w, expert_b):
    b, s, h = x.shape
    xf = x.reshape(-1, h)
    bs = xf.shape[0]
    n_blk = 4
    d3 = _router_d(xf, router_w, router_b, n_blk)
    toks, gates, rows = _make_sc_select(bs, h)(d3.reshape(bs), xf)
    yzero = _zero_fill(d3, bs, h, 4)
    eout = _expert_mm(gates, rows, expert_w, expert_b)
    y = _scatter_rows(toks, eout, yzero)
    return y.reshape(b, s, h)
